# baseline (speedup 1.0000x reference)
import functools

import jax
import jax.numpy as jnp
from jax import lax
from jax.experimental import pallas as pl
from jax.experimental.pallas import tpu as pltpu
from jax.experimental.pallas import tpu_sc as plsc

_NUM_WORKERS = 32
_NUM_SEG = 2
_CHUNK = 200
_BE = 3200


def kernel(V_set, E_set, node_ids):
    V = V_set[0]
    M, De = E_set.shape[1], E_set.shape[2]
    D = V.shape[1]
    E_t = E_set[0].T
    idx = node_ids[0].astype(jnp.int32)
    segM = M // _NUM_SEG
    b_per_w = segM // _NUM_WORKERS
    n_chunks = b_per_w // _CHUNK
    n_pairs = n_chunks // 2
    nb_seg = segM // _BE

    mesh = plsc.VectorSubcoreMesh(core_axis_name="c", subcore_axis_name="s")

    def make_gather(seg):
        @functools.partial(
            pl.kernel,
            mesh=mesh,
            out_type=jax.ShapeDtypeStruct((segM, D), jnp.float32),
            scratch_types=[
                pltpu.VMEM((_CHUNK,), jnp.int32),
                pltpu.VMEM((_CHUNK,), jnp.int32),
                pltpu.VMEM((_CHUNK, D), jnp.float32),
                pltpu.VMEM((_CHUNK, D), jnp.float32),
                pltpu.SemaphoreType.DMA,
                pltpu.SemaphoreType.DMA,
                pltpu.SemaphoreType.DMA,
                pltpu.SemaphoreType.DMA,
            ],
            compiler_params=pltpu.CompilerParams(use_tc_tiling_on_sc=False),
        )
        def _gather(v_hbm, idx_hbm, g_hbm, idx0, idx1, rows0, rows1,
                    g0, g1, o0, o1):
            wid = lax.axis_index("s") * 2 + lax.axis_index("c")
            base = seg * segM + wid * b_per_w
            obase = wid * b_per_w
            idxs = (idx0, idx1)
            rows = (rows0, rows1)
            sg = (g0, g1)
            so = (o0, o1)

            def gather_start(c, b):
                pltpu.sync_copy(idx_hbm.at[pl.ds(base + c * _CHUNK, _CHUNK)],
                                idxs[b])
                pltpu.make_async_copy(v_hbm.at[idxs[b]], rows[b],
                                      sg[b]).start()

            def gather_wait(b):
                pltpu.make_async_copy(v_hbm.at[idxs[b]], rows[b],
                                      sg[b]).wait()

            def out_start(c, b):
                pltpu.make_async_copy(
                    rows[b], g_hbm.at[pl.ds(obase + c * _CHUNK, _CHUNK)],
                    so[b]).start()

            def out_wait(b):
                pltpu.make_async_copy(
                    rows[b], g_hbm.at[pl.ds(0, _CHUNK)], so[b]).wait()

            gather_start(0, 0)

            def body(c2, carry):
                for b in (0, 1):
                    c = 2 * c2 + b
                    gather_wait(b)
                    out_start(c, b)
                    nb = 1 - b
                    if b == 0:
                        @pl.when(c2 >= 1)
                        def _():
                            out_wait(nb)
                        gather_start(c + 1, nb)
                    elif n_chunks % 2 == 1:
                        out_wait(nb)
                        gather_start(c + 1, nb)
                    else:
                        @pl.when(c2 < n_pairs - 1)
                        def _():
                            out_wait(nb)
                            gather_start(c + 1, nb)
                return carry

            lax.fori_loop(0, n_pairs, body, 0)
            if n_chunks % 2 == 1:
                gather_wait(0)
                out_start(n_chunks - 1, 0)
                out_wait(1)
                out_wait(0)
            else:
                out_wait(0)
                out_wait(1)

        return _gather

    Gs = [make_gather(s)(V, idx) for s in range(_NUM_SEG)]

    out_t = None
    for s in range(_NUM_SEG):
        def _concat(*refs, _s=s):
            e_ref, g_ref = refs[0], refs[1]
            o_ref = refs[-1]
            o_ref[0:De, :] = e_ref[...]
            eye = (lax.broadcasted_iota(jnp.int32, (D, D), 0) ==
                   lax.broadcasted_iota(jnp.int32, (D, D), 1)
                   ).astype(jnp.float32)
            o_ref[De:De + D, :] = lax.dot_general(
                eye, g_ref[...],
                dimension_numbers=(((1,), (1,)), ((), ())),
                preferred_element_type=jnp.float32,
                precision=lax.Precision.HIGHEST)

        in_specs = [
            pl.BlockSpec((De, _BE), lambda i, _s=s: (0, i + _s * nb_seg)),
            pl.BlockSpec((_BE, D), lambda i: (i, 0)),
        ]
        operands = [E_t, Gs[s]]
        aliases = {}
        if s > 0:
            in_specs.append(pl.BlockSpec(memory_space=pl.ANY))
            operands.append(out_t)
            aliases = {2: 0}
        out_t = pl.pallas_call(
            _concat,
            grid=(nb_seg,),
            in_specs=in_specs,
            out_specs=pl.BlockSpec((De + D, _BE),
                                   lambda i, _s=s: (0, i + _s * nb_seg)),
            out_shape=jax.ShapeDtypeStruct((De + D, M), jnp.float32),
            input_output_aliases=aliases,
        )(*operands)

    return out_t.T[jnp.newaxis]


# speedup vs baseline: 1.0982x; 1.0982x over previous
import functools

import jax
import jax.numpy as jnp
from jax import lax
from jax.experimental import pallas as pl
from jax.experimental.pallas import tpu as pltpu
from jax.experimental.pallas import tpu_sc as plsc

_NUM_WORKERS = 32
_SEGS = (143360, 176640)
_CHUNKS = (320, 240)
_BE = 2560


def kernel(V_set, E_set, node_ids):
    V = V_set[0]
    M, De = E_set.shape[1], E_set.shape[2]
    D = V.shape[1]
    E_t = E_set[0].T
    idx = node_ids[0].astype(jnp.int32)
    seg_starts = [sum(_SEGS[:i]) for i in range(len(_SEGS))]

    mesh = plsc.VectorSubcoreMesh(core_axis_name="c", subcore_axis_name="s")

    def make_gather(seg):
        segM = _SEGS[seg]
        seg_start = seg_starts[seg]
        chunk = _CHUNKS[seg]
        b_per_w = segM // _NUM_WORKERS
        n_chunks = b_per_w // chunk
        n_pairs = n_chunks // 2
        @functools.partial(
            pl.kernel,
            mesh=mesh,
            out_type=jax.ShapeDtypeStruct((segM, D), jnp.float32),
            scratch_types=[
                pltpu.VMEM((chunk,), jnp.int32),
                pltpu.VMEM((chunk,), jnp.int32),
                pltpu.VMEM((chunk, D), jnp.float32),
                pltpu.VMEM((chunk, D), jnp.float32),
                pltpu.SemaphoreType.DMA,
                pltpu.SemaphoreType.DMA,
                pltpu.SemaphoreType.DMA,
                pltpu.SemaphoreType.DMA,
            ],
            compiler_params=pltpu.CompilerParams(use_tc_tiling_on_sc=False),
        )
        def _gather(v_hbm, idx_hbm, g_hbm, idx0, idx1, rows0, rows1,
                    g0, g1, o0, o1):
            wid = lax.axis_index("s") * 2 + lax.axis_index("c")
            base = seg_start + wid * b_per_w
            obase = wid * b_per_w
            idxs = (idx0, idx1)
            rows = (rows0, rows1)
            sg = (g0, g1)
            so = (o0, o1)

            def gather_start(c, b):
                pltpu.sync_copy(idx_hbm.at[pl.ds(base + c * chunk, chunk)],
                                idxs[b])
                pltpu.make_async_copy(v_hbm.at[idxs[b]], rows[b],
                                      sg[b]).start()

            def gather_wait(b):
                pltpu.make_async_copy(v_hbm.at[idxs[b]], rows[b],
                                      sg[b]).wait()

            def out_start(c, b):
                pltpu.make_async_copy(
                    rows[b], g_hbm.at[pl.ds(obase + c * chunk, chunk)],
                    so[b]).start()

            def out_wait(b):
                pltpu.make_async_copy(
                    rows[b], g_hbm.at[pl.ds(0, chunk)], so[b]).wait()

            gather_start(0, 0)

            def body(c2, carry):
                for b in (0, 1):
                    c = 2 * c2 + b
                    gather_wait(b)
                    out_start(c, b)
                    nb = 1 - b
                    if b == 0:
                        @pl.when(c2 >= 1)
                        def _():
                            out_wait(nb)
                        gather_start(c + 1, nb)
                    elif n_chunks % 2 == 1:
                        out_wait(nb)
                        gather_start(c + 1, nb)
                    else:
                        @pl.when(c2 < n_pairs - 1)
                        def _():
                            out_wait(nb)
                            gather_start(c + 1, nb)
                return carry

            lax.fori_loop(0, n_pairs, body, 0)
            if n_chunks % 2 == 1:
                gather_wait(0)
                out_start(n_chunks - 1, 0)
                out_wait(1)
                out_wait(0)
            else:
                out_wait(0)
                out_wait(1)

        return _gather

    Gs = [make_gather(s)(V, idx) for s in range(len(_SEGS))]

    out_t = None
    for s in range(len(_SEGS)):
        nb_seg = _SEGS[s] // _BE
        boff = seg_starts[s] // _BE
        def _concat(*refs, _s=s):
            e_ref, g_ref = refs[0], refs[1]
            o_ref = refs[-1]
            o_ref[0:De, :] = e_ref[...]
            o_ref[De:De + D, :] = g_ref[...].T

        in_specs = [
            pl.BlockSpec((De, _BE), lambda i, _b=boff: (0, i + _b)),
            pl.BlockSpec((_BE, D), lambda i: (i, 0)),
        ]
        operands = [E_t, Gs[s]]
        aliases = {}
        if s > 0:
            in_specs.append(pl.BlockSpec(memory_space=pl.ANY))
            operands.append(out_t)
            aliases = {2: 0}
        out_t = pl.pallas_call(
            _concat,
            grid=(nb_seg,),
            in_specs=in_specs,
            out_specs=pl.BlockSpec((De + D, _BE),
                                   lambda i, _b=boff: (0, i + _b)),
            out_shape=jax.ShapeDtypeStruct((De + D, M), jnp.float32),
            input_output_aliases=aliases,
        )(*operands)

    return out_t.T[jnp.newaxis]


# speedup vs baseline: 1.1389x; 1.0371x over previous
import functools

import jax
import jax.numpy as jnp
from jax import lax
from jax.experimental import pallas as pl
from jax.experimental.pallas import tpu as pltpu
from jax.experimental.pallas import tpu_sc as plsc

_NUM_WORKERS = 32
_SEGS = (143360, 176640)
_CHUNKS = (320, 240)
_BE = 2560


def kernel(V_set, E_set, node_ids):
    V = V_set[0]
    M, De = E_set.shape[1], E_set.shape[2]
    D = V.shape[1]
    E_t = E_set[0].T
    idxp = jnp.pad(node_ids.astype(jnp.int32), ((0, 0), (0, (-M) % 1024)))
    seg_starts = [sum(_SEGS[:i]) for i in range(len(_SEGS))]

    mesh = plsc.VectorSubcoreMesh(core_axis_name="c", subcore_axis_name="s")

    def make_gather(seg):
        segM = _SEGS[seg]
        seg_start = seg_starts[seg]
        chunk = _CHUNKS[seg]
        b_per_w = segM // _NUM_WORKERS
        n_chunks = b_per_w // chunk
        n_pairs = n_chunks // 2
        @functools.partial(
            pl.kernel,
            mesh=mesh,
            out_type=jax.ShapeDtypeStruct((segM, D), jnp.float32),
            scratch_types=[
                pltpu.VMEM((chunk,), jnp.int32),
                pltpu.VMEM((chunk,), jnp.int32),
                pltpu.VMEM((chunk, D), jnp.float32),
                pltpu.VMEM((chunk, D), jnp.float32),
                pltpu.SemaphoreType.DMA,
                pltpu.SemaphoreType.DMA,
                pltpu.SemaphoreType.DMA,
                pltpu.SemaphoreType.DMA,
            ],
            compiler_params=pltpu.CompilerParams(use_tc_tiling_on_sc=False),
        )
        def _gather(v_hbm, idx_hbm, g_hbm, idx0, idx1, rows0, rows1,
                    g0, g1, o0, o1):
            wid = lax.axis_index("s") * 2 + lax.axis_index("c")
            base = seg_start + wid * b_per_w
            obase = wid * b_per_w
            idxs = (idx0, idx1)
            rows = (rows0, rows1)
            sg = (g0, g1)
            so = (o0, o1)

            def gather_start(c, b):
                pltpu.sync_copy(
                    idx_hbm.at[0, pl.ds(base + c * chunk, chunk)], idxs[b])
                pltpu.make_async_copy(v_hbm.at[idxs[b]], rows[b],
                                      sg[b]).start()

            def gather_wait(b):
                pltpu.make_async_copy(v_hbm.at[idxs[b]], rows[b],
                                      sg[b]).wait()

            def out_start(c, b):
                pltpu.make_async_copy(
                    rows[b], g_hbm.at[pl.ds(obase + c * chunk, chunk)],
                    so[b]).start()

            def out_wait(b):
                pltpu.make_async_copy(
                    rows[b], g_hbm.at[pl.ds(0, chunk)], so[b]).wait()

            gather_start(0, 0)

            def body(c2, carry):
                for b in (0, 1):
                    c = 2 * c2 + b
                    gather_wait(b)
                    out_start(c, b)
                    nb = 1 - b
                    if b == 0:
                        @pl.when(c2 >= 1)
                        def _():
                            out_wait(nb)
                        gather_start(c + 1, nb)
                    elif n_chunks % 2 == 1:
                        out_wait(nb)
                        gather_start(c + 1, nb)
                    else:
                        @pl.when(c2 < n_pairs - 1)
                        def _():
                            out_wait(nb)
                            gather_start(c + 1, nb)
                return carry

            lax.fori_loop(0, n_pairs, body, 0)
            if n_chunks % 2 == 1:
                gather_wait(0)
                out_start(n_chunks - 1, 0)
                out_wait(1)
                out_wait(0)
            else:
                out_wait(0)
                out_wait(1)

        return _gather

    Gs = [make_gather(s)(V, idxp) for s in range(len(_SEGS))]

    out_t = None
    for s in range(len(_SEGS)):
        nb_seg = _SEGS[s] // _BE
        boff = seg_starts[s] // _BE
        def _concat(*refs, _s=s):
            e_ref, g_ref = refs[0], refs[1]
            o_ref = refs[-1]
            o_ref[0:De, :] = e_ref[...]
            o_ref[De:De + D, :] = g_ref[...].T

        in_specs = [
            pl.BlockSpec((De, _BE), lambda i, _b=boff: (0, i + _b)),
            pl.BlockSpec((_BE, D), lambda i: (i, 0)),
        ]
        operands = [E_t, Gs[s]]
        aliases = {}
        if s > 0:
            in_specs.append(pl.BlockSpec(memory_space=pl.ANY))
            operands.append(out_t)
            aliases = {2: 0}
        out_t = pl.pallas_call(
            _concat,
            grid=(nb_seg,),
            in_specs=in_specs,
            out_specs=pl.BlockSpec((De + D, _BE),
                                   lambda i, _b=boff: (0, i + _b)),
            out_shape=jax.ShapeDtypeStruct((De + D, M), jnp.float32),
            input_output_aliases=aliases,
        )(*operands)

    return out_t.T[jnp.newaxis]


# speedup vs baseline: 1.1421x; 1.0028x over previous
import functools

import jax
import jax.numpy as jnp
from jax import lax
from jax.experimental import pallas as pl
from jax.experimental.pallas import tpu as pltpu
from jax.experimental.pallas import tpu_sc as plsc

_NUM_WORKERS = 32
_SEGS = (76800, 102400, 140800)
_CHUNKS = (200, 320, 400)
_BE = 2560


def kernel(V_set, E_set, node_ids):
    V = V_set[0]
    M, De = E_set.shape[1], E_set.shape[2]
    D = V.shape[1]
    E_t = E_set[0].T
    idxp = jnp.pad(node_ids.astype(jnp.int32), ((0, 0), (0, (-M) % 1024)))
    seg_starts = [sum(_SEGS[:i]) for i in range(len(_SEGS))]

    mesh = plsc.VectorSubcoreMesh(core_axis_name="c", subcore_axis_name="s")

    def make_gather(seg):
        segM = _SEGS[seg]
        seg_start = seg_starts[seg]
        chunk = _CHUNKS[seg]
        b_per_w = segM // _NUM_WORKERS
        n_chunks = b_per_w // chunk
        n_pairs = n_chunks // 2
        @functools.partial(
            pl.kernel,
            mesh=mesh,
            out_type=jax.ShapeDtypeStruct((segM, D), jnp.float32),
            scratch_types=[
                pltpu.VMEM((chunk,), jnp.int32),
                pltpu.VMEM((chunk,), jnp.int32),
                pltpu.VMEM((chunk, D), jnp.float32),
                pltpu.VMEM((chunk, D), jnp.float32),
                pltpu.SemaphoreType.DMA,
                pltpu.SemaphoreType.DMA,
                pltpu.SemaphoreType.DMA,
                pltpu.SemaphoreType.DMA,
            ],
            compiler_params=pltpu.CompilerParams(use_tc_tiling_on_sc=False),
        )
        def _gather(v_hbm, idx_hbm, g_hbm, idx0, idx1, rows0, rows1,
                    g0, g1, o0, o1):
            wid = lax.axis_index("s") * 2 + lax.axis_index("c")
            base = seg_start + wid * b_per_w
            obase = wid * b_per_w
            idxs = (idx0, idx1)
            rows = (rows0, rows1)
            sg = (g0, g1)
            so = (o0, o1)

            def gather_start(c, b):
                pltpu.sync_copy(
                    idx_hbm.at[0, pl.ds(base + c * chunk, chunk)], idxs[b])
                pltpu.make_async_copy(v_hbm.at[idxs[b]], rows[b],
                                      sg[b]).start()

            def gather_wait(b):
                pltpu.make_async_copy(v_hbm.at[idxs[b]], rows[b],
                                      sg[b]).wait()

            def out_start(c, b):
                pltpu.make_async_copy(
                    rows[b], g_hbm.at[pl.ds(obase + c * chunk, chunk)],
                    so[b]).start()

            def out_wait(b):
                pltpu.make_async_copy(
                    rows[b], g_hbm.at[pl.ds(0, chunk)], so[b]).wait()

            gather_start(0, 0)

            def body(c2, carry):
                for b in (0, 1):
                    c = 2 * c2 + b
                    gather_wait(b)
                    out_start(c, b)
                    nb = 1 - b
                    if b == 0:
                        @pl.when(c2 >= 1)
                        def _():
                            out_wait(nb)
                        gather_start(c + 1, nb)
                    elif n_chunks % 2 == 1:
                        out_wait(nb)
                        gather_start(c + 1, nb)
                    else:
                        @pl.when(c2 < n_pairs - 1)
                        def _():
                            out_wait(nb)
                            gather_start(c + 1, nb)
                return carry

            lax.fori_loop(0, n_pairs, body, 0)
            if n_chunks % 2 == 1:
                gather_wait(0)
                out_start(n_chunks - 1, 0)
                out_wait(1)
                out_wait(0)
            else:
                out_wait(0)
                out_wait(1)

        return _gather

    Gs = [make_gather(s)(V, idxp) for s in range(len(_SEGS))]

    out_t = None
    for s in range(len(_SEGS)):
        nb_seg = _SEGS[s] // _BE
        boff = seg_starts[s] // _BE
        def _concat(*refs, _s=s):
            e_ref, g_ref = refs[0], refs[1]
            o_ref = refs[-1]
            o_ref[0:De, :] = e_ref[...]
            o_ref[De:De + D, :] = g_ref[...].T

        in_specs = [
            pl.BlockSpec((De, _BE), lambda i, _b=boff: (0, i + _b)),
            pl.BlockSpec((_BE, D), lambda i: (i, 0)),
        ]
        operands = [E_t, Gs[s]]
        aliases = {}
        if s > 0:
            in_specs.append(pl.BlockSpec(memory_space=pl.ANY))
            operands.append(out_t)
            aliases = {2: 0}
        out_t = pl.pallas_call(
            _concat,
            grid=(nb_seg,),
            in_specs=in_specs,
            out_specs=pl.BlockSpec((De + D, _BE),
                                   lambda i, _b=boff: (0, i + _b)),
            out_shape=jax.ShapeDtypeStruct((De + D, M), jnp.float32),
            input_output_aliases=aliases,
        )(*operands)

    return out_t.T[jnp.newaxis]
